# baseline (speedup 1.0000x reference)
import functools

import jax
import jax.numpy as jnp
from jax import lax
from jax.experimental import pallas as pl
from jax.experimental.pallas import tpu as pltpu
from jax.experimental.pallas import tpu_sc as plsc

D = 128
LANES = 16
CHUNK = 128


def _embed_sum(ids, pids, sids, word_emb, pos_emb, sent_emb):
    NW, n_chunks, _ = ids.shape
    N = NW * n_chunks * CHUNK
    info = plsc.get_sparse_core_info()
    NC = info.num_cores
    per_w = n_chunks * CHUNK

    mesh = plsc.VectorSubcoreMesh(core_axis_name="c", subcore_axis_name="s")

    @functools.partial(
        pl.kernel,
        mesh=mesh,
        out_type=jax.ShapeDtypeStruct((N, D), jnp.float32),
        scratch_types=[
            pltpu.VMEM((n_chunks, CHUNK), jnp.int32),
            pltpu.VMEM((n_chunks, CHUNK), jnp.int32),
            pltpu.VMEM((n_chunks, CHUNK), jnp.int32),
            pltpu.VMEM((CHUNK, D), jnp.float32),
            pltpu.VMEM((CHUNK, D), jnp.float32),
            pltpu.VMEM((CHUNK, D), jnp.float32),
            pltpu.VMEM((CHUNK, D), jnp.float32),
            pltpu.VMEM((CHUNK, D), jnp.float32),
            pltpu.VMEM((CHUNK, D), jnp.float32),
        ] + [pltpu.SemaphoreType.DMA] * 8,
    )
    def k(ids_hbm, pids_hbm, sids_hbm, word_hbm, pos_hbm, sent_hbm, out_hbm,
          widx, pidx, sidx, wrows0, wrows1, prows0, prows1, srows0, srows1,
          sem_w0, sem_w1, sem_p0, sem_p1, sem_s0, sem_s1, sem_o0, sem_o1):
        wid = lax.axis_index("s") * NC + lax.axis_index("c")
        base = wid * per_w

        wrows = (wrows0, wrows1)
        prows = (prows0, prows1)
        srows = (srows0, srows1)
        sem_w = (sem_w0, sem_w1)
        sem_p = (sem_p0, sem_p1)
        sem_s = (sem_s0, sem_s1)
        sem_o = (sem_o0, sem_o1)

        pltpu.sync_copy(ids_hbm.at[wid], widx)
        pltpu.sync_copy(pids_hbm.at[wid], pidx)
        pltpu.sync_copy(sids_hbm.at[wid], sidx)

        def start_gathers(c):
            b = c & 1
            cw = pltpu.async_copy(word_hbm.at[widx.at[c]], wrows[b], sem_w[b])
            cp = pltpu.async_copy(pos_hbm.at[pidx.at[c]], prows[b], sem_p[b])
            cs = pltpu.async_copy(sent_hbm.at[sidx.at[c]], srows[b], sem_s[b])
            return cw, cp, cs

        pend = {0: start_gathers(0)}
        out_pend = {}
        for c in range(n_chunks):
            b = c & 1
            if c + 1 < n_chunks:
                if (c - 1) in out_pend:
                    out_pend.pop(c - 1).wait()
                pend[c + 1] = start_gathers(c + 1)
            cw, cp, cs = pend.pop(c)
            cw.wait()
            cp.wait()
            cs.wait()

            def add_body(r, _, b=b):
                for j in range(D // LANES):
                    sl = pl.ds(j * LANES, LANES)
                    v = prows[b][r, sl] + srows[b][r, sl]
                    plsc.addupdate(wrows[b].at[r, sl], v)
                return 0

            lax.fori_loop(0, CHUNK, add_body, 0)
            out_pend[c] = pltpu.async_copy(
                wrows[b], out_hbm.at[pl.ds(base + c * CHUNK, CHUNK)], sem_o[b])
        for c in sorted(out_pend):
            out_pend.pop(c).wait()

    return k(ids, pids, sids, word_emb, pos_emb, sent_emb)


def kernel(input_ids, sent_ids_tensor, position_ids, word_embedding,
           pos_embedding, sent_embedding):
    B, S = input_ids.shape
    N = B * S
    info = plsc.get_sparse_core_info()
    NW = info.num_cores * info.num_subcores
    per_w = N // NW
    n_chunks = per_w // CHUNK
    ids = input_ids.reshape(NW, n_chunks, CHUNK).astype(jnp.int32)
    pids = position_ids.reshape(NW, n_chunks, CHUNK).astype(jnp.int32)
    sids = sent_ids_tensor.reshape(NW, n_chunks, CHUNK).astype(jnp.int32)
    out = _embed_sum(ids, pids, sids, word_embedding, pos_embedding,
                     sent_embedding)
    return out.reshape(B, S, D)


# speedup vs baseline: 5.0874x; 5.0874x over previous
import functools

import jax
import jax.numpy as jnp
from jax import lax
from jax.experimental import pallas as pl
from jax.experimental.pallas import tpu as pltpu
from jax.experimental.pallas import tpu_sc as plsc

D = 128
LANES = 16
CHUNK = 128


def _embed_sum(ids, pids, sids, word_emb, pos_emb, sent_emb):
    NW, n_chunks, _ = ids.shape
    N = NW * n_chunks * CHUNK
    info = plsc.get_sparse_core_info()
    NC = info.num_cores
    per_w = n_chunks * CHUNK

    mesh = plsc.VectorSubcoreMesh(core_axis_name="c", subcore_axis_name="s")

    @functools.partial(
        pl.kernel,
        mesh=mesh,
        out_type=jax.ShapeDtypeStruct((N, D), jnp.float32),
        scratch_types=[
            pltpu.VMEM((n_chunks, CHUNK), jnp.int32),
            pltpu.VMEM((n_chunks, CHUNK), jnp.int32),
            pltpu.VMEM((n_chunks, CHUNK), jnp.int32),
            pltpu.VMEM((CHUNK, D), jnp.float32),
            pltpu.VMEM((CHUNK, D), jnp.float32),
            pltpu.VMEM((CHUNK, D), jnp.float32),
            pltpu.VMEM((CHUNK, D), jnp.float32),
            pltpu.VMEM((CHUNK, D), jnp.float32),
            pltpu.VMEM_SHARED((4096, D), jnp.float32),
            pltpu.VMEM_SHARED((64, D), jnp.float32),
        ] + [pltpu.SemaphoreType.DMA] * 7,
    )
    def k(ids_hbm, pids_hbm, sids_hbm, word_hbm, pos_hbm, sent_hbm, out_hbm,
          widx, pidx, sidx, wrows0, wrows1, prows0, prows1, srows1,
          pos_sh, sent_sh, sem_w0, sem_w1, sem_p0, sem_p1, sem_s0,
          sem_o0, sem_o1):
        wid = lax.axis_index("s") * NC + lax.axis_index("c")
        base = wid * per_w

        wrows = (wrows0, wrows1)
        prows = (prows0, prows1)
        srows = srows1
        sem_w = (sem_w0, sem_w1)
        sem_p = (sem_p0, sem_p1)
        sem_s = sem_s0
        sem_o = (sem_o0, sem_o1)

        sid_ax = lax.axis_index("s")
        rows_per_tile = pos_hbm.shape[0] // 16
        pltpu.sync_copy(
            pos_hbm.at[pl.ds(sid_ax * rows_per_tile, rows_per_tile)],
            pos_sh.at[pl.ds(sid_ax * rows_per_tile, rows_per_tile)])
        pltpu.sync_copy(sent_hbm, sent_sh.at[pl.ds(sid_ax * 4, 4)])

        pltpu.sync_copy(ids_hbm.at[wid], widx)
        pltpu.sync_copy(pids_hbm.at[wid], pidx)
        pltpu.sync_copy(sids_hbm.at[wid], sidx)

        soff = jnp.full((LANES,), 4, jnp.int32) * sid_ax
        for cc in range(n_chunks):
            for jj in range(CHUNK // LANES):
                ssl = pl.ds(jj * LANES, LANES)
                sidx[cc, ssl] = sidx[cc, ssl] + soff

        plsc.subcore_barrier()

        def start_gathers(c):
            b = c & 1
            cw = pltpu.async_copy(word_hbm.at[widx.at[c]], wrows[b], sem_w[b])
            cp = pltpu.async_copy(pos_sh.at[pidx.at[c]], prows[b], sem_p[b])
            return cw, cp

        def start_sent(c):
            return pltpu.async_copy(sent_sh.at[sidx.at[c]], srows, sem_s)

        pend = {0: start_gathers(0)}
        cs_pend = start_sent(0)
        out_pend = {}
        for c in range(n_chunks):
            b = c & 1
            if c + 1 < n_chunks:
                if (c - 1) in out_pend:
                    out_pend.pop(c - 1).wait()
                pend[c + 1] = start_gathers(c + 1)
            cw, cp = pend.pop(c)
            cw.wait()
            cp.wait()
            cs_pend.wait()

            def add_body(r, _, b=b):
                for j in range(D // LANES):
                    sl = pl.ds(j * LANES, LANES)
                    v = prows[b][r, sl] + srows[r, sl]
                    plsc.addupdate(wrows[b].at[r, sl], v)
                return 0

            lax.fori_loop(0, CHUNK, add_body, 0)
            if c + 1 < n_chunks:
                cs_pend = start_sent(c + 1)
            out_pend[c] = pltpu.async_copy(
                wrows[b], out_hbm.at[pl.ds(base + c * CHUNK, CHUNK)], sem_o[b])
        for c in sorted(out_pend):
            out_pend.pop(c).wait()

    return k(ids, pids, sids, word_emb, pos_emb, sent_emb)


def kernel(input_ids, sent_ids_tensor, position_ids, word_embedding,
           pos_embedding, sent_embedding):
    B, S = input_ids.shape
    N = B * S
    info = plsc.get_sparse_core_info()
    NW = info.num_cores * info.num_subcores
    per_w = N // NW
    n_chunks = per_w // CHUNK
    ids = input_ids.reshape(NW, n_chunks, CHUNK).astype(jnp.int32)
    pids = position_ids.reshape(NW, n_chunks, CHUNK).astype(jnp.int32)
    sids = sent_ids_tensor.reshape(NW, n_chunks, CHUNK).astype(jnp.int32)
    out = _embed_sum(ids, pids, sids, word_embedding, pos_embedding,
                     sent_embedding)
    return out.reshape(B, S, D)
